# baseline (speedup 1.0000x reference)
import functools

import jax
import jax.numpy as jnp
from jax import lax
from jax.experimental import pallas as pl
from jax.experimental.pallas import tpu as pltpu
from jax.experimental.pallas import tpu_sc as plsc

_EPS = 1e-12


def _sc_gather(word_embeddings, flat_ids, n_tokens, hid):
    info = plsc.get_sparse_core_info()
    nw = info.num_cores * info.num_subcores
    per_w = n_tokens // nw
    chunk = 64
    n_chunks = per_w // chunk
    mesh = plsc.VectorSubcoreMesh(core_axis_name="c", subcore_axis_name="s")

    @functools.partial(
        pl.kernel,
        mesh=mesh,
        out_type=jax.ShapeDtypeStruct((n_tokens, hid), jnp.float32),
        scratch_types=[
            pltpu.VMEM((per_w,), jnp.int32),
            pltpu.VMEM((chunk, hid), jnp.float32),
            pltpu.VMEM((chunk, hid), jnp.float32),
            pltpu.SemaphoreType.DMA,
            pltpu.SemaphoreType.DMA,
        ],
    )
    def gather_k(table_hbm, idx_hbm, out_hbm, idx_v, buf0, buf1, gsem, wsem):
        wid = lax.axis_index("s") * info.num_cores + lax.axis_index("c")
        base = wid * per_w
        bufs = (buf0, buf1)
        pltpu.sync_copy(idx_hbm.at[pl.ds(base, per_w)], idx_v)

        def start_gather(i):
            return pltpu.async_copy(
                table_hbm.at[idx_v.at[pl.ds(i * chunk, chunk)]], bufs[i % 2], gsem
            )

        gathers = [start_gather(0), start_gather(1)]
        writes = [None, None]
        for i in range(n_chunks):
            gathers[i % 2].wait()
            writes[i % 2] = pltpu.async_copy(
                bufs[i % 2], out_hbm.at[pl.ds(base + i * chunk, chunk)], wsem
            )
            if i + 2 < n_chunks:
                writes[i % 2].wait()
                gathers[i % 2] = start_gather(i + 2)
        writes[(n_chunks - 2) % 2].wait()
        writes[(n_chunks - 1) % 2].wait()

    return gather_k(word_embeddings, flat_ids)


def _ln_body(g_ref, pos_ref, tok_ref, w_ref, b_ref, o_ref):
    hid = g_ref.shape[-1]
    x = g_ref[0] + pos_ref[...] + tok_ref[...]
    ones = jnp.ones((hid, 1), jnp.float32)
    s1 = jax.lax.dot(x, ones, preferred_element_type=jnp.float32)
    s2 = jax.lax.dot(x * x, ones, preferred_element_type=jnp.float32)
    mean = s1 * (1.0 / hid)
    var = s2 * (1.0 / hid) - mean * mean
    inv = lax.rsqrt(var + _EPS)
    o_ref[0] = ((x - mean) * inv) * w_ref[...] + b_ref[...]


def kernel(input_ids, word_embeddings, position_embeddings, token_type_embeddings, ln_weight, ln_bias):
    b, s = input_ids.shape
    vocab, hid = word_embeddings.shape
    n_tokens = b * s
    flat_ids = input_ids.reshape(n_tokens).astype(jnp.int32)

    gathered = _sc_gather(word_embeddings, flat_ids, n_tokens, hid)
    gathered = gathered.reshape(b, s, hid)

    bs = 1024
    out = pl.pallas_call(
        _ln_body,
        grid=(b, s // bs),
        in_specs=[
            pl.BlockSpec((1, bs, hid), lambda i, j: (i, j, 0)),
            pl.BlockSpec((bs, hid), lambda i, j: (j, 0)),
            pl.BlockSpec((1, hid), lambda i, j: (0, 0)),
            pl.BlockSpec((1, hid), lambda i, j: (0, 0)),
            pl.BlockSpec((1, hid), lambda i, j: (0, 0)),
        ],
        out_specs=pl.BlockSpec((1, bs, hid), lambda i, j: (i, j, 0)),
        out_shape=jax.ShapeDtypeStruct((b, s, hid), jnp.float32),
    )(
        gathered,
        position_embeddings,
        token_type_embeddings[0:1],
        ln_weight.reshape(1, hid),
        ln_bias.reshape(1, hid),
    )
    return out


# speedup vs baseline: 1.0905x; 1.0905x over previous
import functools

import jax
import jax.numpy as jnp
from jax import lax
from jax.experimental import pallas as pl
from jax.experimental.pallas import tpu as pltpu
from jax.experimental.pallas import tpu_sc as plsc

_EPS = 1e-12


def _sc_gather(word_embeddings, flat_ids, n_tokens, hid):
    info = plsc.get_sparse_core_info()
    nw = info.num_cores * info.num_subcores
    per_w = n_tokens // nw
    chunk = 64
    n_chunks = per_w // chunk
    mesh = plsc.VectorSubcoreMesh(core_axis_name="c", subcore_axis_name="s")

    @functools.partial(
        pl.kernel,
        mesh=mesh,
        out_type=jax.ShapeDtypeStruct((n_tokens, hid), jnp.float32),
        scratch_types=[
            pltpu.VMEM((per_w,), jnp.int32),
            pltpu.VMEM((chunk, hid), jnp.float32),
            pltpu.VMEM((chunk, hid), jnp.float32),
            pltpu.SemaphoreType.DMA,
            pltpu.SemaphoreType.DMA,
        ],
    )
    def gather_k(table_hbm, idx_hbm, out_hbm, idx_v, buf0, buf1, gsem, wsem):
        wid = lax.axis_index("s") * info.num_cores + lax.axis_index("c")
        base = wid * per_w
        bufs = (buf0, buf1)
        pltpu.sync_copy(idx_hbm.at[pl.ds(base, per_w)], idx_v)

        def start_gather(i):
            return pltpu.async_copy(
                table_hbm.at[idx_v.at[pl.ds(i * chunk, chunk)]], bufs[i % 2], gsem
            )

        gathers = [start_gather(0), start_gather(1)]
        writes = [None, None]
        for i in range(n_chunks):
            gathers[i % 2].wait()
            writes[i % 2] = pltpu.async_copy(
                bufs[i % 2], out_hbm.at[pl.ds(base + i * chunk, chunk)], wsem
            )
            if i + 2 < n_chunks:
                writes[i % 2].wait()
                gathers[i % 2] = start_gather(i + 2)
        writes[(n_chunks - 2) % 2].wait()
        writes[(n_chunks - 1) % 2].wait()

    return gather_k(word_embeddings, flat_ids)


def _ln_body(g_ref, pos_ref, tok_ref, w_ref, b_ref, o_ref):
    x = g_ref[0] + pos_ref[...] + tok_ref[...]
    mean = jnp.mean(x, axis=-1, keepdims=True)
    xc = x - mean
    var = jnp.mean(xc * xc, axis=-1, keepdims=True)
    o_ref[0] = (xc * lax.rsqrt(var + _EPS)) * w_ref[...] + b_ref[...]


def kernel(input_ids, word_embeddings, position_embeddings, token_type_embeddings, ln_weight, ln_bias):
    b, s = input_ids.shape
    vocab, hid = word_embeddings.shape
    n_tokens = b * s
    flat_ids = input_ids.reshape(n_tokens).astype(jnp.int32)

    gathered = _sc_gather(word_embeddings, flat_ids, n_tokens, hid)
    gathered = gathered.reshape(b, s, hid)

    bs = 2048
    out = pl.pallas_call(
        _ln_body,
        grid=(b, s // bs),
        in_specs=[
            pl.BlockSpec((1, bs, hid), lambda i, j: (i, j, 0)),
            pl.BlockSpec((bs, hid), lambda i, j: (j, 0)),
            pl.BlockSpec((1, hid), lambda i, j: (0, 0)),
            pl.BlockSpec((1, hid), lambda i, j: (0, 0)),
            pl.BlockSpec((1, hid), lambda i, j: (0, 0)),
        ],
        out_specs=pl.BlockSpec((1, bs, hid), lambda i, j: (i, j, 0)),
        out_shape=jax.ShapeDtypeStruct((b, s, hid), jnp.float32),
    )(
        gathered,
        position_embeddings,
        token_type_embeddings[0:1],
        ln_weight.reshape(1, hid),
        ln_bias.reshape(1, hid),
    )
    return out
